# baseline (speedup 1.0000x reference)
import jax
import jax.numpy as jnp
from jax import lax
from jax.experimental import pallas as pl
from jax.experimental.pallas import tpu as pltpu
from jax.experimental.pallas import tpu_sc as plsc

_N_USER = 25000
_N_ITEM = 25000
_N = _N_USER + _N_ITEM
_NNZ = 800000
_D = 64
_HD = _D // 2
_BATCH = 4096

_NC = 2
_NS = 16

_EB = 512
_CH = 128
_NGRP = 99
_NNZ_PAD = _NS * _NGRP * _EB
_GPT = _NGRP * _EB

_ROWS_PER_TILE = _N // _NS

_MESH = plsc.VectorSubcoreMesh(
    core_axis_name="c", subcore_axis_name="s", num_cores=_NC, num_subcores=_NS)


def _spmm_body(ego2, rows2_h, cols_h, vals_h, side_c,
               acc, gath, colsb, valsb, sidx, gidx, msem, gsem, ssem):
  c = lax.axis_index("c")
  s = lax.axis_index("s")

  @pl.loop(0, _EB)
  def _zero_gath(i):
    for j in range(2):
      gath[i, pl.ds(16 * j, 16)] = jnp.zeros((16,), jnp.float32)

  base_t = s * _ROWS_PER_TILE

  @pl.loop(0, 6)
  def _zero_acc(i):
    pltpu.sync_copy(gath.at[pl.ds(0, 512)], acc.at[pl.ds(base_t + 512 * i, 512)])

  pltpu.sync_copy(gath.at[pl.ds(0, 53)], acc.at[pl.ds(base_t + 3072, 53)])

  plsc.subcore_barrier()

  ebase_t = s * _GPT

  def stage(G, m):
    gb = ebase_t + G * _EB
    d1 = pltpu.async_copy(cols_h.at[pl.ds(gb, _EB)], colsb.at[m], msem)
    d2 = pltpu.async_copy(vals_h.at[pl.ds(gb, _EB)], valsb.at[m], msem)
    d3 = pltpu.async_copy(rows2_h.at[pl.ds(s * 4 * _NGRP + G * 4, 4)],
                          sidx.at[m], msem)
    return (d1, d2, d3)

  def drain_stage(G, m):
    gb = ebase_t + G * _EB
    pltpu.make_async_copy(cols_h.at[pl.ds(gb, _EB)], colsb.at[m], msem).wait()
    pltpu.make_async_copy(vals_h.at[pl.ds(gb, _EB)], valsb.at[m], msem).wait()
    pltpu.make_async_copy(rows2_h.at[pl.ds(s * 4 * _NGRP + G * 4, 4)],
                          sidx.at[m], msem).wait()

  def compute_gidx(m):
    for i in range(32):
      gidx[m, i // 8, pl.ds((i % 8) * 16, 16)] = (
          colsb[m, pl.ds(16 * i, 16)] * 2 + c)

  def fire_gather(m, j):
    pltpu.async_copy(ego2.at[gidx.at[m, j]], gath.at[pl.ds(128 * j, 128)],
                     gsem)

  def wait_gather(m, j):
    pltpu.make_async_copy(ego2.at[gidx.at[m, j]], gath.at[pl.ds(128 * j, 128)],
                          gsem).wait()

  def fire_scatter(m, j):
    pltpu.async_copy(gath.at[pl.ds(128 * j, 128)], acc.at[sidx.at[m, j]],
                     ssem, add=True)

  def drain_scatter(m, j):
    pltpu.make_async_copy(gath.at[pl.ds(128 * j, 128)], acc.at[sidx.at[m, j]],
                          ssem).wait()

  def scale(m, j):
    @plsc.parallel_loop(128 * j, 128 * (j + 1), step=16, unroll=2)
    def _scale(eb):
      vv = valsb[m, pl.ds(eb, 16)]
      for k in range(16):
        v = jnp.broadcast_to(vv[k], (16,))
        gath[eb + k, pl.ds(0, 16)] = gath[eb + k, pl.ds(0, 16)] * v
        gath[eb + k, pl.ds(16, 16)] = gath[eb + k, pl.ds(16, 16)] * v

  for d in stage(0, 0) + stage(1, 1):
    d.wait()
  compute_gidx(0)
  compute_gidx(1)
  fire_gather(0, 0)
  fire_gather(0, 1)

  @pl.loop(0, _NGRP, step=3)
  def _groups(g):
    for gg in range(3):
      G = g + gg
      for j in range(4):
        kk = 4 * G + j
        if j == 0:
          @pl.when((G >= 1) & (G + 1 < _NGRP))
          def _():
            drain_stage(G + 1, (gg + 1) % 3)
            compute_gidx((gg + 1) % 3)
        if j == 2:
          @pl.when(G + 2 < _NGRP)
          def _():
            stage(G + 2, (gg + 2) % 3)
        pj, pm = (j - 2, gg) if j >= 2 else (j + 2, (gg - 1) % 3)
        @pl.when(kk >= 2)
        def _():
          drain_scatter(pm, pj)
        wait_gather(gg, j)
        scale(gg, j)
        fire_scatter(gg, j)
        nj, nm = (j + 2, gg) if j < 2 else (j - 2, (gg + 1) % 3)
        @pl.when(kk + 2 < 4 * _NGRP)
        def _():
          fire_gather(nm, nj)

  drain_scatter(2, 2)
  drain_scatter(2, 3)

  plsc.subcore_barrier()

  @pl.loop(0, 6)
  def _writeout(i):
    r0 = base_t + 512 * i
    pltpu.sync_copy(acc.at[pl.ds(r0, 512)], gath.at[pl.ds(0, 512)])
    pltpu.sync_copy(gath.at[pl.ds(0, 512)], side_c.at[c, pl.ds(r0, 512)])

  r1 = base_t + 3072
  pltpu.sync_copy(acc.at[pl.ds(r1, 53)], gath.at[pl.ds(0, 53)])
  pltpu.sync_copy(gath.at[pl.ds(0, 53)], side_c.at[c, pl.ds(r1, 53)])


_spmm = pl.kernel(
    _spmm_body,
    out_type=jax.ShapeDtypeStruct((_NC, _N, _HD), jnp.float32),
    mesh=_MESH,
    scratch_types=[
        pltpu.VMEM_SHARED((_N, _HD), jnp.float32),
        pltpu.VMEM((_EB, _HD), jnp.float32),
        pltpu.VMEM((3, _EB), jnp.int32),
        pltpu.VMEM((3, _EB), jnp.float32),
        pltpu.VMEM((3, 4, 128), jnp.int32),
        pltpu.VMEM((3, 4, 128), jnp.int32),
        pltpu.SemaphoreType.DMA,
        pltpu.SemaphoreType.DMA,
        pltpu.SemaphoreType.DMA,
    ],
    compiler_params=pltpu.CompilerParams(use_tc_tiling_on_sc=False),
)


_RB = 2000


def _dense_body(ego_ref, slo_ref, shi_ref, wgc_ref, bgc_ref, wbi_ref, bbi_ref,
                ego_out, norm_out):
  ego = ego_ref[...]
  side = jnp.concatenate([slo_ref[0], shi_ref[0]], axis=1)
  dn = (((1,), (0,)), ((), ()))
  su = lax.dot_general(side, wgc_ref[...], dn,
                       precision=lax.Precision.HIGHEST,
                       preferred_element_type=jnp.float32) + bgc_ref[...]
  bi = lax.dot_general(ego * side, wbi_ref[...], dn,
                       precision=lax.Precision.HIGHEST,
                       preferred_element_type=jnp.float32) + bbi_ref[...]
  e = su + bi
  e = jnp.where(e >= 0, e, 0.2 * e)
  nrm = jnp.sqrt(jnp.sum(e * e, axis=1, keepdims=True))
  ego_out[...] = e
  norm_out[...] = e / jnp.maximum(nrm, 1e-12)


def _dense(ego, side_c, wgc, bgc, wbi, bbi):
  bs = pl.BlockSpec((_RB, _D), lambda i: (i, 0))
  slo = pl.BlockSpec((1, _RB, _HD), lambda i: (0, i, 0))
  shi = pl.BlockSpec((1, _RB, _HD), lambda i: (1, i, 0))
  ws = pl.BlockSpec((_D, _D), lambda i: (0, 0))
  vs = pl.BlockSpec((1, _D), lambda i: (0, 0))
  return pl.pallas_call(
      _dense_body,
      grid=(_N // _RB,),
      in_specs=[bs, slo, shi, ws, vs, ws, vs],
      out_specs=[bs, bs],
      out_shape=[jax.ShapeDtypeStruct((_N, _D), jnp.float32)] * 2,
  )(ego, side_c, side_c, wgc, bgc, wbi, bbi)


def _gather_body(e0, n1, n2, n3, users, pos, neg, u_out, p_out, n_out,
                 idxb, rowb, sem):
  c = lax.axis_index("c")
  s = lax.axis_index("s")
  base = (s * _NC + c) * (_BATCH // (_NC * _NS))
  tabs = (e0, n1, n2, n3)
  for idx_h, off, out in ((users, 0, u_out), (pos, _N_USER, p_out),
                          (neg, _N_USER, n_out)):
    pltpu.sync_copy(idx_h.at[pl.ds(base, 128)], idxb)
    if off:
      for i in range(8):
        idxb[pl.ds(16 * i, 16)] = idxb[pl.ds(16 * i, 16)] + off
    for t in range(4):
      pltpu.async_copy(tabs[t].at[idxb], rowb, sem).wait()
      pltpu.sync_copy(rowb, out.at[pl.ds(base, 128), pl.ds(64 * t, 64)])


_batch_gather = pl.kernel(
    _gather_body,
    out_type=(jax.ShapeDtypeStruct((_BATCH, 4 * _D), jnp.float32),) * 3,
    mesh=_MESH,
    scratch_types=[
        pltpu.VMEM((128,), jnp.int32),
        pltpu.VMEM((128, _D), jnp.float32),
        pltpu.SemaphoreType.DMA,
    ],
    compiler_params=pltpu.CompilerParams(use_tc_tiling_on_sc=False),
)


def kernel(user_emb, item_emb,
           W_gc_0, b_gc_0, W_bi_0, b_bi_0,
           W_gc_1, b_gc_1, W_bi_1, b_bi_1,
           W_gc_2, b_gc_2, W_bi_2, b_bi_2,
           adj_rows, adj_cols, adj_vals,
           users, pos_items, neg_items):
  ego = jnp.concatenate([user_emb, item_emb], axis=0)
  e0 = ego
  layer_params = [
      (W_gc_0, b_gc_0, W_bi_0, b_bi_0),
      (W_gc_1, b_gc_1, W_bi_1, b_bi_1),
      (W_gc_2, b_gc_2, W_bi_2, b_bi_2),
  ]
  pad = _NNZ_PAD - _NNZ
  rows_p = jnp.concatenate([adj_rows, jnp.zeros((pad,), jnp.int32)])
  cols_p = jnp.concatenate([adj_cols, jnp.zeros((pad,), jnp.int32)])
  vals_p = jnp.concatenate([adj_vals, jnp.zeros((pad,), jnp.float32)])
  rows2 = rows_p.reshape(_NNZ_PAD // 128, 128)
  normed = []
  for wgc, bgc, wbi, bbi in layer_params:
    side_c = _spmm(ego.reshape(2 * _N, _HD), rows2, cols_p, vals_p)
    ego, nm = _dense(ego, side_c, wgc, bgc, wbi, bbi)
    normed.append(nm)
  return _batch_gather(e0, normed[0], normed[1], normed[2],
                       users, pos_items, neg_items)


# speedup vs baseline: 1.0693x; 1.0693x over previous
import jax
import jax.numpy as jnp
from jax import lax
from jax.experimental import pallas as pl
from jax.experimental.pallas import tpu as pltpu
from jax.experimental.pallas import tpu_sc as plsc

_N_USER = 25000
_N_ITEM = 25000
_N = _N_USER + _N_ITEM
_NNZ = 800000
_D = 64
_HD = _D // 2
_BATCH = 4096

_NC = 2
_NS = 16

_EB = 512
_CH = 128
_NGRP = 99
_NNZ_PAD = _NS * _NGRP * _EB
_GPT = _NGRP * _EB

_ROWS_PER_TILE = _N // _NS

_MESH = plsc.VectorSubcoreMesh(
    core_axis_name="c", subcore_axis_name="s", num_cores=_NC, num_subcores=_NS)


def _spmm_body(ego2, rows2_h, cols_h, vals_h, side_c,
               acc, gath, colsb, valsb, sidx, gidx, msem, gsem, ssem):
  c = lax.axis_index("c")
  s = lax.axis_index("s")

  @pl.loop(0, _EB)
  def _zero_gath(i):
    for j in range(2):
      gath[i, pl.ds(16 * j, 16)] = jnp.zeros((16,), jnp.float32)

  base_t = s * _ROWS_PER_TILE

  @pl.loop(0, 6)
  def _zero_acc(i):
    pltpu.sync_copy(gath.at[pl.ds(0, 512)], acc.at[pl.ds(base_t + 512 * i, 512)])

  pltpu.sync_copy(gath.at[pl.ds(0, 53)], acc.at[pl.ds(base_t + 3072, 53)])

  plsc.subcore_barrier()

  ebase_t = s * _GPT

  def stage(G, m):
    gb = ebase_t + G * _EB
    d1 = pltpu.async_copy(cols_h.at[pl.ds(gb, _EB)], colsb.at[m], msem)
    d2 = pltpu.async_copy(vals_h.at[pl.ds(gb, _EB)], valsb.at[m], msem)
    d3 = pltpu.async_copy(rows2_h.at[pl.ds(s * 4 * _NGRP + G * 4, 4)],
                          sidx.at[m], msem)
    return (d1, d2, d3)

  def drain_stage(G, m):
    gb = ebase_t + G * _EB
    pltpu.make_async_copy(cols_h.at[pl.ds(gb, _EB)], colsb.at[m], msem).wait()
    pltpu.make_async_copy(vals_h.at[pl.ds(gb, _EB)], valsb.at[m], msem).wait()
    pltpu.make_async_copy(rows2_h.at[pl.ds(s * 4 * _NGRP + G * 4, 4)],
                          sidx.at[m], msem).wait()

  def compute_gidx(m):
    for i in range(32):
      gidx[m, i // 8, pl.ds((i % 8) * 16, 16)] = (
          colsb[m, pl.ds(16 * i, 16)] * 2 + c)

  def fire_gather(m, j):
    pltpu.async_copy(ego2.at[gidx.at[m, j]], gath.at[pl.ds(128 * j, 128)],
                     gsem)

  def wait_gather(m, j):
    pltpu.make_async_copy(ego2.at[gidx.at[m, j]], gath.at[pl.ds(128 * j, 128)],
                          gsem).wait()

  def fire_scatter(m, j):
    pltpu.async_copy(gath.at[pl.ds(128 * j, 128)], acc.at[sidx.at[m, j]],
                     ssem, add=True)

  def drain_scatter(m, j):
    pltpu.make_async_copy(gath.at[pl.ds(128 * j, 128)], acc.at[sidx.at[m, j]],
                          ssem).wait()

  def scale(m, j):
    @plsc.parallel_loop(128 * j, 128 * (j + 1), step=16)
    def _scale(eb):
      vv = valsb[m, pl.ds(eb, 16)]
      for k in range(16):
        v = jnp.broadcast_to(vv[k], (16,))
        gath[eb + k, pl.ds(0, 16)] = gath[eb + k, pl.ds(0, 16)] * v
        gath[eb + k, pl.ds(16, 16)] = gath[eb + k, pl.ds(16, 16)] * v

  for d in stage(0, 0) + stage(1, 1):
    d.wait()
  compute_gidx(0)
  compute_gidx(1)
  fire_gather(0, 0)
  fire_gather(0, 1)
  fire_gather(0, 2)

  @pl.loop(0, _NGRP, step=3)
  def _groups(g):
    for gg in range(3):
      G = g + gg
      for j in range(4):
        kk = 4 * G + j
        if j == 0:
          @pl.when((G >= 1) & (G + 1 < _NGRP))
          def _():
            drain_stage(G + 1, (gg + 1) % 3)
            compute_gidx((gg + 1) % 3)
        if j == 2:
          @pl.when(G + 2 < _NGRP)
          def _():
            stage(G + 2, (gg + 2) % 3)
        pj, pm = (j - 1, gg) if j >= 1 else (3, (gg - 1) % 3)
        @pl.when(kk >= 1)
        def _():
          drain_scatter(pm, pj)
        wait_gather(gg, j)
        scale(gg, j)
        fire_scatter(gg, j)
        nj, nm = (j + 3, gg) if j < 1 else (j - 1, (gg + 1) % 3)
        @pl.when(kk + 3 < 4 * _NGRP)
        def _():
          fire_gather(nm, nj)

  drain_scatter(2, 3)

  plsc.subcore_barrier()

  @pl.loop(0, 6)
  def _writeout(i):
    r0 = base_t + 512 * i
    pltpu.sync_copy(acc.at[pl.ds(r0, 512)], gath.at[pl.ds(0, 512)])
    pltpu.sync_copy(gath.at[pl.ds(0, 512)], side_c.at[c, pl.ds(r0, 512)])

  r1 = base_t + 3072
  pltpu.sync_copy(acc.at[pl.ds(r1, 53)], gath.at[pl.ds(0, 53)])
  pltpu.sync_copy(gath.at[pl.ds(0, 53)], side_c.at[c, pl.ds(r1, 53)])


_spmm = pl.kernel(
    _spmm_body,
    out_type=jax.ShapeDtypeStruct((_NC, _N, _HD), jnp.float32),
    mesh=_MESH,
    scratch_types=[
        pltpu.VMEM_SHARED((_N, _HD), jnp.float32),
        pltpu.VMEM((_EB, _HD), jnp.float32),
        pltpu.VMEM((3, _EB), jnp.int32),
        pltpu.VMEM((3, _EB), jnp.float32),
        pltpu.VMEM((3, 4, 128), jnp.int32),
        pltpu.VMEM((3, 4, 128), jnp.int32),
        pltpu.SemaphoreType.DMA,
        pltpu.SemaphoreType.DMA,
        pltpu.SemaphoreType.DMA,
    ],
    compiler_params=pltpu.CompilerParams(use_tc_tiling_on_sc=False),
)


_RB = 2000


def _dense_body(ego_ref, slo_ref, shi_ref, wgc_ref, bgc_ref, wbi_ref, bbi_ref,
                ego_out, norm_out):
  ego = ego_ref[...]
  side = jnp.concatenate([slo_ref[0], shi_ref[0]], axis=1)
  dn = (((1,), (0,)), ((), ()))
  su = lax.dot_general(side, wgc_ref[...], dn,
                       precision=lax.Precision.HIGHEST,
                       preferred_element_type=jnp.float32) + bgc_ref[...]
  bi = lax.dot_general(ego * side, wbi_ref[...], dn,
                       precision=lax.Precision.HIGHEST,
                       preferred_element_type=jnp.float32) + bbi_ref[...]
  e = su + bi
  e = jnp.where(e >= 0, e, 0.2 * e)
  nrm = jnp.sqrt(jnp.sum(e * e, axis=1, keepdims=True))
  ego_out[...] = e
  norm_out[...] = e / jnp.maximum(nrm, 1e-12)


def _dense(ego, side_c, wgc, bgc, wbi, bbi):
  bs = pl.BlockSpec((_RB, _D), lambda i: (i, 0))
  slo = pl.BlockSpec((1, _RB, _HD), lambda i: (0, i, 0))
  shi = pl.BlockSpec((1, _RB, _HD), lambda i: (1, i, 0))
  ws = pl.BlockSpec((_D, _D), lambda i: (0, 0))
  vs = pl.BlockSpec((1, _D), lambda i: (0, 0))
  return pl.pallas_call(
      _dense_body,
      grid=(_N // _RB,),
      in_specs=[bs, slo, shi, ws, vs, ws, vs],
      out_specs=[bs, bs],
      out_shape=[jax.ShapeDtypeStruct((_N, _D), jnp.float32)] * 2,
  )(ego, side_c, side_c, wgc, bgc, wbi, bbi)


def _gather_body(e0, n1, n2, n3, users, pos, neg, u_out, p_out, n_out,
                 idxb, rowb, sem):
  c = lax.axis_index("c")
  s = lax.axis_index("s")
  base = (s * _NC + c) * (_BATCH // (_NC * _NS))
  tabs = (e0, n1, n2, n3)
  for idx_h, off, out in ((users, 0, u_out), (pos, _N_USER, p_out),
                          (neg, _N_USER, n_out)):
    pltpu.sync_copy(idx_h.at[pl.ds(base, 128)], idxb)
    if off:
      for i in range(8):
        idxb[pl.ds(16 * i, 16)] = idxb[pl.ds(16 * i, 16)] + off
    for t in range(4):
      pltpu.async_copy(tabs[t].at[idxb], rowb, sem).wait()
      pltpu.sync_copy(rowb, out.at[pl.ds(base, 128), pl.ds(64 * t, 64)])


_batch_gather = pl.kernel(
    _gather_body,
    out_type=(jax.ShapeDtypeStruct((_BATCH, 4 * _D), jnp.float32),) * 3,
    mesh=_MESH,
    scratch_types=[
        pltpu.VMEM((128,), jnp.int32),
        pltpu.VMEM((128, _D), jnp.float32),
        pltpu.SemaphoreType.DMA,
    ],
    compiler_params=pltpu.CompilerParams(use_tc_tiling_on_sc=False),
)


def kernel(user_emb, item_emb,
           W_gc_0, b_gc_0, W_bi_0, b_bi_0,
           W_gc_1, b_gc_1, W_bi_1, b_bi_1,
           W_gc_2, b_gc_2, W_bi_2, b_bi_2,
           adj_rows, adj_cols, adj_vals,
           users, pos_items, neg_items):
  ego = jnp.concatenate([user_emb, item_emb], axis=0)
  e0 = ego
  layer_params = [
      (W_gc_0, b_gc_0, W_bi_0, b_bi_0),
      (W_gc_1, b_gc_1, W_bi_1, b_bi_1),
      (W_gc_2, b_gc_2, W_bi_2, b_bi_2),
  ]
  pad = _NNZ_PAD - _NNZ
  rows_p = jnp.concatenate([adj_rows, jnp.zeros((pad,), jnp.int32)])
  cols_p = jnp.concatenate([adj_cols, jnp.zeros((pad,), jnp.int32)])
  vals_p = jnp.concatenate([adj_vals, jnp.zeros((pad,), jnp.float32)])
  rows2 = rows_p.reshape(_NNZ_PAD // 128, 128)
  normed = []
  for wgc, bgc, wbi, bbi in layer_params:
    side_c = _spmm(ego.reshape(2 * _N, _HD), rows2, cols_p, vals_p)
    ego, nm = _dense(ego, side_c, wgc, bgc, wbi, bbi)
    normed.append(nm)
  return _batch_gather(e0, normed[0], normed[1], normed[2],
                       users, pos_items, neg_items)
